# Initial kernel scaffold; baseline (speedup 1.0000x reference)
import jax
import jax.numpy as jnp
from jax.experimental import pallas as pl


def kernel(x, edge_index, w1a, w1b, w2a, w2b):
    raise NotImplementedError("write your pallas kernel here")


# speedup vs baseline: 7.0282x; 7.0282x over previous
import functools

import jax
import jax.numpy as jnp
from jax import lax
from jax.experimental import pallas as pl
from jax.experimental.pallas import tpu as pltpu
from jax.experimental.pallas import tpu_sc as plsc

N_NODES = 10000
D = 128
N_EDGES = 320000
NC = 2
NS = 16
NW = NC * NS
EPT = N_EDGES // NW
K = 80
CHUNKS = EPT // K
STRIPE = 624
TAIL0 = NS * STRIPE
TAIL = N_NODES - TAIL0


def _seg_sum_partials(x, src, dst, zeros):
    mesh = plsc.VectorSubcoreMesh(core_axis_name="c", subcore_axis_name="s")

    @functools.partial(
        pl.kernel,
        out_type=jax.ShapeDtypeStruct((NC, N_NODES, D), jnp.float32),
        mesh=mesh,
        scratch_types=[
            pltpu.VMEM((CHUNKS, K), jnp.int32),
            pltpu.VMEM((CHUNKS, K), jnp.int32),
            pltpu.VMEM((K, D), jnp.float32),
            pltpu.VMEM_SHARED((N_NODES, D), jnp.float32),
            pltpu.SemaphoreType.DMA,
        ],
    )
    def body(x_hbm, src_hbm, dst_hbm, zero_hbm, out_hbm,
             src_v, dst_v, rows_v, acc_sh, sem):
        c = lax.axis_index("c")
        s = lax.axis_index("s")
        wid = s * NC + c
        pltpu.sync_copy(src_hbm.at[wid], src_v)
        pltpu.sync_copy(dst_hbm.at[wid], dst_v)
        r0 = s * STRIPE
        pltpu.sync_copy(zero_hbm.at[pl.ds(r0, STRIPE)],
                        acc_sh.at[pl.ds(r0, STRIPE)])

        @pl.when(s == NS - 1)
        def _():
            pltpu.sync_copy(zero_hbm.at[pl.ds(TAIL0, TAIL)],
                            acc_sh.at[pl.ds(TAIL0, TAIL)])

        plsc.subcore_barrier()

        def step(j, carry):
            pltpu.async_copy(x_hbm.at[src_v.at[j]], rows_v, sem).wait()
            pltpu.sync_copy(rows_v, acc_sh.at[dst_v.at[j]], add=True)
            return carry

        lax.fori_loop(0, CHUNKS, step, 0)
        plsc.subcore_barrier()
        pltpu.sync_copy(acc_sh.at[pl.ds(r0, STRIPE)],
                        out_hbm.at[c, pl.ds(r0, STRIPE)])

        @pl.when(s == NS - 1)
        def _():
            pltpu.sync_copy(acc_sh.at[pl.ds(TAIL0, TAIL)],
                            out_hbm.at[c, pl.ds(TAIL0, TAIL)])

    return body(x, src, dst, zeros)


def _mlp(x, p0, p1, wa, wb, normalize):
    BR = 1000
    grid = (N_NODES // BR,)

    def body(x_b, p0_b, p1_b, wa_b, wb_b, o_b):
        h = x_b[...] + p0_b[...] + p1_b[...]
        h = jnp.dot(h, wa_b[...], preferred_element_type=jnp.float32)
        h = jnp.maximum(h, 0.0)
        h = jnp.dot(h, wb_b[...], preferred_element_type=jnp.float32)
        if normalize:
            n = jnp.sqrt(jnp.sum(h * h, axis=1, keepdims=True))
            h = h / jnp.maximum(n, 1e-12)
        o_b[...] = h

    return pl.pallas_call(
        body,
        grid=grid,
        in_specs=[
            pl.BlockSpec((BR, D), lambda i: (i, 0)),
            pl.BlockSpec((BR, D), lambda i: (i, 0)),
            pl.BlockSpec((BR, D), lambda i: (i, 0)),
            pl.BlockSpec((D, D), lambda i: (0, 0)),
            pl.BlockSpec((D, D), lambda i: (0, 0)),
        ],
        out_specs=pl.BlockSpec((BR, D), lambda i: (i, 0)),
        out_shape=jax.ShapeDtypeStruct((N_NODES, D), jnp.float32),
    )(x, p0, p1, wa, wb)


def kernel(x, edge_index, w1a, w1b, w2a, w2b):
    ei = edge_index.astype(jnp.int32)
    src = ei[0].reshape(NW, CHUNKS, K)
    dst = ei[1].reshape(NW, CHUNKS, K)
    zeros = jnp.zeros((N_NODES, D), jnp.float32)

    p = _seg_sum_partials(x, src, dst, zeros)
    h1 = _mlp(x, p[0], p[1], w1a, w1b, normalize=False)
    q = _seg_sum_partials(h1, src, dst, zeros)
    return _mlp(h1, q[0], q[1], w2a, w2b, normalize=True)
